# baseline (speedup 1.0000x reference)
import functools

import jax
import jax.numpy as jnp
from jax import lax
from jax.experimental import pallas as pl
from jax.experimental.pallas import tpu as pltpu
from jax.experimental.pallas import tpu_sc as plsc

_B, _N, _C = 4, 8192, 128
_S, _K = 1024, 32
_TS = 256

_NW = 32
_PPW = (_B * _S) // _NW


_RF = 4
_R = _B * _RF
_L = _N // _RF


def _fold4(v):
    return jnp.concatenate([v, v, v, v], axis=0)


def _comb4(v, op):
    return op(op(v[0:_B], v[_B:2 * _B]), op(v[2 * _B:3 * _B], v[3 * _B:]))


def _fps_kernel(xyzs_ref, far_ref, idx_ref, nxyz_ref):
    lane = lax.broadcasted_iota(jnp.int32, (_R, _L), 1)
    rowo = (lax.broadcasted_iota(jnp.int32, (_R, _L), 0) // _B) * _L
    iota_f = (lane + rowo).astype(jnp.float32)
    iota48_f = jnp.concatenate([iota_f, iota_f, iota_f], axis=0)
    xyzs = xyzs_ref[...]
    xs = xyzs[0:_R]
    ys = xyzs[_R:2 * _R]
    zs = xyzs[2 * _R:3 * _R]

    iota_s16 = lax.broadcasted_iota(jnp.int32, (_R, _S), 1)

    def body(i, carry):
        dist, fc, acc = carry
        fc48 = jnp.concatenate([_fold4(fc)] * 3, axis=0)
        neg = jnp.float32(-1e30)
        red = jnp.max(jnp.where(iota48_f == fc48, xyzs, neg),
                      axis=1, keepdims=True)
        cx = _comb4(red[0:_R], jnp.maximum)
        cy = _comb4(red[_R:2 * _R], jnp.maximum)
        cz = _comb4(red[2 * _R:3 * _R], jnp.maximum)
        vals16 = jnp.concatenate([fc, cx, cy, cz], axis=0)
        acc = jnp.where(iota_s16 == i, vals16, acc)
        cx16 = _fold4(cx)
        cy16 = _fold4(cy)
        cz16 = _fold4(cz)
        dx = xs - cx16
        dy = ys - cy16
        dz = zs - cz16
        d = dx * dx + dy * dy + dz * dz
        dist = jnp.minimum(dist, d)
        m = _comb4(jnp.max(dist, axis=1, keepdims=True), jnp.maximum)
        cand = jnp.where(dist == _fold4(m), iota_f, jnp.float32(_N))
        fc = _comb4(jnp.min(cand, axis=1, keepdims=True), jnp.minimum)
        return dist, fc, acc

    dist0 = jnp.full((_R, _L), 1e10, dtype=jnp.float32)
    acc0 = jnp.zeros((_R, _S), dtype=jnp.float32)
    _, _, acc = lax.fori_loop(
        0, _S, body, (dist0, far_ref[...].astype(jnp.float32), acc0))
    idx_ref[...] = acc[0:_B].astype(jnp.int32)
    nxyz_ref[:, :, 0:1] = acc[_B:2 * _B][:, :, None]
    nxyz_ref[:, :, 1:2] = acc[2 * _B:3 * _B][:, :, None]
    nxyz_ref[:, :, 2:3] = acc[3 * _B:][:, :, None]


def _fps(xyzs, far):
    return pl.pallas_call(
        _fps_kernel,
        out_shape=(
            jax.ShapeDtypeStruct((_B, _S), jnp.int32),
            jax.ShapeDtypeStruct((_B, _S, 3), jnp.float32),
        ),
    )(xyzs, far)


def _bf16_round(v):
    u = jax.lax.bitcast_convert_type(v, jnp.uint32)
    r = ((u >> 16) & jnp.uint32(1)) + jnp.uint32(0x7FFF)
    u = (u + r) & jnp.uint32(0xFFFF0000)
    return jax.lax.bitcast_convert_type(u, jnp.float32)


def _topk_kernel(nxyz_ref, xs_ref, ys_ref, zs_ref, idx_ref):
    sx = nxyz_ref[0, :, 0:1]
    sy = nxyz_ref[0, :, 1:2]
    sz = nxyz_ref[0, :, 2:3]
    dxr = xs_ref[0]
    dyr = ys_ref[0]
    dzr = zs_ref[0]
    d2 = dxr * dxr + dyr * dyr + dzr * dzr
    s2 = sx * sx + sy * sy + sz * sz
    cross = (_bf16_round(sx) * _bf16_round(dxr)
             + _bf16_round(sy) * _bf16_round(dyr)
             + _bf16_round(sz) * _bf16_round(dzr))
    dist = ((-2.0) * cross + s2) + d2
    iota_f = lax.broadcasted_iota(jnp.int32, (_TS, _N), 1).astype(jnp.float32)
    big = jnp.float32(3e38)
    sels = []
    for k in range(_K):
        m = jnp.min(dist, axis=1, keepdims=True)
        sel = jnp.min(jnp.where(dist == m, iota_f, jnp.float32(_N)),
                      axis=1, keepdims=True)
        sels.append(sel)
        dist = jnp.where(iota_f == sel, big, dist)
    idx_ref[0] = jnp.concatenate(sels, axis=1).astype(jnp.int32)


def _topk(new_xyz, xs3, ys3, zs3):
    ns = new_xyz.shape[1]
    return pl.pallas_call(
        _topk_kernel,
        grid=(_B, ns // _TS),
        in_specs=[
            pl.BlockSpec((1, _TS, 3), lambda b, s: (b, s, 0)),
            pl.BlockSpec((1, 1, _N), lambda b, s: (b, 0, 0)),
            pl.BlockSpec((1, 1, _N), lambda b, s: (b, 0, 0)),
            pl.BlockSpec((1, 1, _N), lambda b, s: (b, 0, 0)),
        ],
        out_specs=pl.BlockSpec((1, _TS, _K), lambda b, s: (b, s, 0)),
        out_shape=jax.ShapeDtypeStruct((_B, ns, _K), jnp.int32),
    )(new_xyz, xs3, ys3, zs3)


def _make_edge_body(ppw):
    def _edge_body(x_hbm, nbr_hbm, ctr_hbm, out_hbm,
                   idxn_v, idxc_v, ctr_rows, buf0, buf1, semc, semg0, semg1):
        w = lax.axis_index("c") * 16 + lax.axis_index("s")
        base = w * ppw
        pltpu.sync_copy(nbr_hbm.at[pl.ds(base, ppw)], idxn_v)
        pltpu.sync_copy(ctr_hbm.at[pl.ds(base, ppw)], idxc_v)
        pltpu.async_copy(x_hbm.at[idxc_v], ctr_rows, semc).wait()

        def do_pair(p, buf, semg):
            pltpu.make_async_copy(x_hbm.at[idxn_v.at[p]], buf, semg).wait()
            cvs = [ctr_rows[p, pl.ds(16 * j, 16)] for j in range(_C // 16)]

            def krow(k, c2):
                for j in range(_C // 16):
                    sl = pl.ds(16 * j, 16)
                    buf[k, sl] = buf[k, sl] - cvs[j]
                return c2

            lax.fori_loop(0, _K, krow, 0)
            pltpu.sync_copy(buf, out_hbm.at[pl.ds((base + p) * _K, _K)])

        pltpu.async_copy(x_hbm.at[idxn_v.at[0]], buf0, semg0)

        def step(g, carry):
            p0 = 2 * g
            p1 = 2 * g + 1
            pltpu.async_copy(x_hbm.at[idxn_v.at[p1]], buf1, semg1)
            do_pair(p0, buf0, semg0)
            pn = jnp.minimum(p1 + 1, ppw - 1)
            pltpu.async_copy(x_hbm.at[idxn_v.at[pn]], buf0, semg0)
            do_pair(p1, buf1, semg1)
            return carry

        lax.fori_loop(0, ppw // 2, step, 0)
        pltpu.make_async_copy(
            x_hbm.at[idxn_v.at[ppw - 1]], buf0, semg0).wait()

    return _edge_body


def _edge(xflat, nbr_flat, ctr_flat):
    npairs = nbr_flat.shape[0]
    ppw = npairs // _NW
    mesh = plsc.VectorSubcoreMesh(core_axis_name="c", subcore_axis_name="s")
    f = pl.kernel(
        _make_edge_body(ppw),
        out_type=jax.ShapeDtypeStruct((npairs * _K, _C), jnp.float32),
        mesh=mesh,
        scratch_types=[
            pltpu.VMEM((ppw, _K), jnp.int32),
            pltpu.VMEM((ppw,), jnp.int32),
            pltpu.VMEM((ppw, _C), jnp.float32),
            pltpu.VMEM((_K, _C), jnp.float32),
            pltpu.VMEM((_K, _C), jnp.float32),
            pltpu.SemaphoreType.DMA,
            pltpu.SemaphoreType.DMA,
            pltpu.SemaphoreType.DMA,
        ],
    )
    return f(xflat, nbr_flat, ctr_flat)


def kernel(xyz, x, farthest):
    xs = xyz[:, :, 0]
    ys = xyz[:, :, 1]
    zs = xyz[:, :, 2]

    def fold(p):
        return p.reshape(_B, _RF, _L).transpose(1, 0, 2).reshape(_R, _L)

    xyzs = jnp.concatenate([fold(xs), fold(ys), fold(zs)], axis=0)
    far = farthest[:, None].astype(jnp.int32)
    fps_idx, new_xyz = _fps(xyzs, far)

    boff = (jnp.arange(_B, dtype=jnp.int32) * _N)
    fps_flat = fps_idx + boff[:, None]
    xflat = x.reshape(_B * _N, _C)
    xs3 = xs[:, None, :]
    ys3 = ys[:, None, :]
    zs3 = zs[:, None, :]

    nchunk = 1
    half = _S // nchunk
    edges = []
    for ci in range(nchunk):
        sl = slice(ci * half, (ci + 1) * half)
        nbr = _topk(new_xyz[:, sl], xs3, ys3, zs3)
        nbr_flat = (nbr + boff[:, None, None]).reshape(_B * half, _K)
        ctr_flat = fps_flat[:, sl].reshape(_B * half)
        edges.append(
            _edge(xflat, nbr_flat, ctr_flat).reshape(_B, half, _K, _C))
    edge = jnp.concatenate(edges, axis=1)
    return new_xyz, edge


# speedup vs baseline: 1.0724x; 1.0724x over previous
import functools

import jax
import jax.numpy as jnp
from jax import lax
from jax.experimental import pallas as pl
from jax.experimental.pallas import tpu as pltpu
from jax.experimental.pallas import tpu_sc as plsc

_B, _N, _C = 4, 8192, 128
_S, _K = 1024, 32
_TS = 128

_NW = 32
_PPW = (_B * _S) // _NW


_RF = 4
_R = _B * _RF
_L = _N // _RF


def _fold4(v):
    return jnp.concatenate([v, v, v, v], axis=0)


def _comb4(v, op):
    return op(op(v[0:_B], v[_B:2 * _B]), op(v[2 * _B:3 * _B], v[3 * _B:]))


def _fps_kernel(xyzs_ref, far_ref, idx_ref, nxyz_ref):
    lane = lax.broadcasted_iota(jnp.int32, (_R, _L), 1)
    rowo = (lax.broadcasted_iota(jnp.int32, (_R, _L), 0) // _B) * _L
    iota_f = (lane + rowo).astype(jnp.float32)
    iota48_f = jnp.concatenate([iota_f, iota_f, iota_f], axis=0)
    xyzs = xyzs_ref[...]
    xs = xyzs[0:_R]
    ys = xyzs[_R:2 * _R]
    zs = xyzs[2 * _R:3 * _R]

    iota_s16 = lax.broadcasted_iota(jnp.int32, (_R, _S), 1)

    def body(i, carry):
        dist, fc, acc = carry
        fc48 = jnp.concatenate([_fold4(fc)] * 3, axis=0)
        neg = jnp.float32(-1e30)
        red = jnp.max(jnp.where(iota48_f == fc48, xyzs, neg),
                      axis=1, keepdims=True)
        cx = _comb4(red[0:_R], jnp.maximum)
        cy = _comb4(red[_R:2 * _R], jnp.maximum)
        cz = _comb4(red[2 * _R:3 * _R], jnp.maximum)
        vals16 = jnp.concatenate([fc, cx, cy, cz], axis=0)
        acc = jnp.where(iota_s16 == i, vals16, acc)
        cx16 = _fold4(cx)
        cy16 = _fold4(cy)
        cz16 = _fold4(cz)
        dx = xs - cx16
        dy = ys - cy16
        dz = zs - cz16
        d = dx * dx + dy * dy + dz * dz
        dist = jnp.minimum(dist, d)
        m = _comb4(jnp.max(dist, axis=1, keepdims=True), jnp.maximum)
        cand = jnp.where(dist == _fold4(m), iota_f, jnp.float32(_N))
        fc = _comb4(jnp.min(cand, axis=1, keepdims=True), jnp.minimum)
        return dist, fc, acc

    dist0 = jnp.full((_R, _L), 1e10, dtype=jnp.float32)
    acc0 = jnp.zeros((_R, _S), dtype=jnp.float32)
    _, _, acc = lax.fori_loop(
        0, _S, body, (dist0, far_ref[...].astype(jnp.float32), acc0))
    idx_ref[...] = acc[0:_B].astype(jnp.int32)
    nxyz_ref[:, :, 0:1] = acc[_B:2 * _B][:, :, None]
    nxyz_ref[:, :, 1:2] = acc[2 * _B:3 * _B][:, :, None]
    nxyz_ref[:, :, 2:3] = acc[3 * _B:][:, :, None]


def _fps(xyzs, far):
    return pl.pallas_call(
        _fps_kernel,
        out_shape=(
            jax.ShapeDtypeStruct((_B, _S), jnp.int32),
            jax.ShapeDtypeStruct((_B, _S, 3), jnp.float32),
        ),
    )(xyzs, far)


def _bf16_round(v):
    u = jax.lax.bitcast_convert_type(v, jnp.uint32)
    r = ((u >> 16) & jnp.uint32(1)) + jnp.uint32(0x7FFF)
    u = (u + r) & jnp.uint32(0xFFFF0000)
    return jax.lax.bitcast_convert_type(u, jnp.float32)


def _topk_kernel(nxyz_ref, xs_ref, ys_ref, zs_ref, idx_ref):
    sx = nxyz_ref[0, :, 0:1]
    sy = nxyz_ref[0, :, 1:2]
    sz = nxyz_ref[0, :, 2:3]
    dxr = xs_ref[0]
    dyr = ys_ref[0]
    dzr = zs_ref[0]
    d2 = dxr * dxr + dyr * dyr + dzr * dzr
    s2 = sx * sx + sy * sy + sz * sz
    cross = (_bf16_round(sx) * _bf16_round(dxr)
             + _bf16_round(sy) * _bf16_round(dyr)
             + _bf16_round(sz) * _bf16_round(dzr))
    dist = ((-2.0) * cross + s2) + d2
    iota_f = lax.broadcasted_iota(jnp.int32, (_TS, _N), 1).astype(jnp.float32)
    big = jnp.float32(3e38)
    sels = []
    for k in range(_K):
        m = jnp.min(dist, axis=1, keepdims=True)
        sel = jnp.min(jnp.where(dist == m, iota_f, jnp.float32(_N)),
                      axis=1, keepdims=True)
        sels.append(sel)
        dist = jnp.where(iota_f == sel, big, dist)
    idx_ref[0] = jnp.concatenate(sels, axis=1).astype(jnp.int32)


def _topk(new_xyz, xs3, ys3, zs3):
    ns = new_xyz.shape[1]
    return pl.pallas_call(
        _topk_kernel,
        grid=(_B, ns // _TS),
        in_specs=[
            pl.BlockSpec((1, _TS, 3), lambda b, s: (b, s, 0)),
            pl.BlockSpec((1, 1, _N), lambda b, s: (b, 0, 0)),
            pl.BlockSpec((1, 1, _N), lambda b, s: (b, 0, 0)),
            pl.BlockSpec((1, 1, _N), lambda b, s: (b, 0, 0)),
        ],
        out_specs=pl.BlockSpec((1, _TS, _K), lambda b, s: (b, s, 0)),
        out_shape=jax.ShapeDtypeStruct((_B, ns, _K), jnp.int32),
    )(new_xyz, xs3, ys3, zs3)


def _make_edge_body(ppw):
    def _edge_body(x_hbm, nbr_hbm, ctr_hbm, out_hbm,
                   idxn_v, idxc_v, ctr_rows, buf0, buf1, semc, semg0, semg1):
        w = lax.axis_index("c") * 16 + lax.axis_index("s")
        base = w * ppw
        pltpu.sync_copy(nbr_hbm.at[pl.ds(base, ppw)], idxn_v)
        pltpu.sync_copy(ctr_hbm.at[pl.ds(base, ppw)], idxc_v)
        pltpu.async_copy(x_hbm.at[idxc_v], ctr_rows, semc).wait()

        def do_pair(p, buf, semg):
            pltpu.make_async_copy(x_hbm.at[idxn_v.at[p]], buf, semg).wait()
            cvs = [ctr_rows[p, pl.ds(16 * j, 16)] for j in range(_C // 16)]

            def krow(k, c2):
                for j in range(_C // 16):
                    sl = pl.ds(16 * j, 16)
                    buf[k, sl] = buf[k, sl] - cvs[j]
                return c2

            lax.fori_loop(0, _K, krow, 0)
            pltpu.sync_copy(buf, out_hbm.at[pl.ds((base + p) * _K, _K)])

        pltpu.async_copy(x_hbm.at[idxn_v.at[0]], buf0, semg0)

        def step(g, carry):
            p0 = 2 * g
            p1 = 2 * g + 1
            pltpu.async_copy(x_hbm.at[idxn_v.at[p1]], buf1, semg1)
            do_pair(p0, buf0, semg0)
            pn = jnp.minimum(p1 + 1, ppw - 1)
            pltpu.async_copy(x_hbm.at[idxn_v.at[pn]], buf0, semg0)
            do_pair(p1, buf1, semg1)
            return carry

        lax.fori_loop(0, ppw // 2, step, 0)
        pltpu.make_async_copy(
            x_hbm.at[idxn_v.at[ppw - 1]], buf0, semg0).wait()

    return _edge_body


def _edge(xflat, nbr_flat, ctr_flat):
    npairs = nbr_flat.shape[0]
    ppw = npairs // _NW
    mesh = plsc.VectorSubcoreMesh(core_axis_name="c", subcore_axis_name="s")
    f = pl.kernel(
        _make_edge_body(ppw),
        out_type=jax.ShapeDtypeStruct((npairs * _K, _C), jnp.float32),
        mesh=mesh,
        scratch_types=[
            pltpu.VMEM((ppw, _K), jnp.int32),
            pltpu.VMEM((ppw,), jnp.int32),
            pltpu.VMEM((ppw, _C), jnp.float32),
            pltpu.VMEM((_K, _C), jnp.float32),
            pltpu.VMEM((_K, _C), jnp.float32),
            pltpu.SemaphoreType.DMA,
            pltpu.SemaphoreType.DMA,
            pltpu.SemaphoreType.DMA,
        ],
    )
    return f(xflat, nbr_flat, ctr_flat)


def kernel(xyz, x, farthest):
    xs = xyz[:, :, 0]
    ys = xyz[:, :, 1]
    zs = xyz[:, :, 2]

    def fold(p):
        return p.reshape(_B, _RF, _L).transpose(1, 0, 2).reshape(_R, _L)

    xyzs = jnp.concatenate([fold(xs), fold(ys), fold(zs)], axis=0)
    far = farthest[:, None].astype(jnp.int32)
    fps_idx, new_xyz = _fps(xyzs, far)

    boff = (jnp.arange(_B, dtype=jnp.int32) * _N)
    fps_flat = fps_idx + boff[:, None]
    xflat = x.reshape(_B * _N, _C)
    xs3 = xs[:, None, :]
    ys3 = ys[:, None, :]
    zs3 = zs[:, None, :]

    nchunk = 1
    half = _S // nchunk
    edges = []
    for ci in range(nchunk):
        sl = slice(ci * half, (ci + 1) * half)
        nbr = _topk(new_xyz[:, sl], xs3, ys3, zs3)
        nbr_flat = (nbr + boff[:, None, None]).reshape(_B * half, _K)
        ctr_flat = fps_flat[:, sl].reshape(_B * half)
        edges.append(
            _edge(xflat, nbr_flat, ctr_flat).reshape(_B, half, _K, _C))
    edge = jnp.concatenate(edges, axis=1)
    return new_xyz, edge
